# baseline (speedup 1.0000x reference)
import functools

import jax
import jax.numpy as jnp
from jax import lax
from jax.experimental import pallas as pl
from jax.experimental.pallas import tpu as pltpu
from jax.experimental.pallas import tpu_sc as plsc

N_U = 12500
N_I = 37500
N = N_U + N_I
D = 64
H = 32
E = 800000
B = 4096
NC = 2
NS = 16
G = 128
NCHUNKS = E // G
SA = 3128
SA_LAST = N - (NS - 1) * SA
NBUF = 4
SB = 32
NSB = 12
CH_FLOOR = NCHUNKS // NS
CH_EXTRA = NCHUNKS % NS

_mesh = plsc.VectorSubcoreMesh(core_axis_name="c", subcore_axis_name="s")

_GATHER_DNUMS = lax.GatherDimensionNumbers(
    offset_dims=(), collapsed_slice_dims=(0,), start_index_map=(0,))


def _bcast_lane(vec, j):
    idx = jnp.full((16, 1), j, jnp.int32)
    return lax.gather(vec, idx, _GATHER_DNUMS, (1,),
                      mode=lax.GatherScatterMode.PROMISE_IN_BOUNDS)


@functools.partial(
    pl.kernel,
    out_type=jax.ShapeDtypeStruct((2 * N, H), jnp.float32),
    mesh=_mesh,
    scratch_types=[
        pltpu.VMEM_SHARED((N, H), jnp.float32),
        pltpu.VMEM((SB, G), jnp.int32),
        pltpu.VMEM((SB, G), jnp.int32),
        pltpu.VMEM((SB, G), jnp.float32),
        pltpu.VMEM((G, H), jnp.float32),
        pltpu.VMEM((G, H), jnp.float32),
        pltpu.VMEM((G, H), jnp.float32),
        pltpu.VMEM((G, H), jnp.float32),
        pltpu.SemaphoreType.DMA((NBUF,)),
        pltpu.SemaphoreType.DMA((NBUF,)),
    ],
    compiler_params=pltpu.CompilerParams(use_tc_tiling_on_sc=False, needs_layout_passes=False),
)
def _prop(cur, col2d, row2d, val2d, zeros, out, acc, colv, rowv, valv,
          msg0, msg1, msg2, msg3, gsem, ssem):
    c = lax.axis_index("c")
    s = lax.axis_index("s")
    msgs = (msg0, msg1, msg2, msg3)
    r0 = pl.multiple_of(s * SA, 8)

    @pl.when(s < NS - 1)
    def _():
        pltpu.sync_copy(zeros.at[pl.ds(r0, SA)], acc.at[pl.ds(r0, SA)])

    @pl.when(s == NS - 1)
    def _():
        pltpu.sync_copy(zeros.at[pl.ds((NS - 1) * SA, SA_LAST)],
                        acc.at[pl.ds((NS - 1) * SA, SA_LAST)])

    plsc.subcore_barrier()

    cb = s * CH_FLOOR + jnp.minimum(s, CH_EXTRA)
    nch = jnp.where(s < CH_EXTRA, CH_FLOOR + 1, CH_FLOOR)

    def sb_body(sb, carry):
        crow = cb + sb * SB
        pltpu.sync_copy(col2d.at[pl.ds(c * NCHUNKS + crow, SB)], colv)
        pltpu.sync_copy(row2d.at[pl.ds(crow, SB)], rowv)
        pltpu.sync_copy(val2d.at[pl.ds(crow, SB)], valv)

        def blk_body(blk, carry2):
            gd = []
            for b in range(NBUF):
                ci = blk * NBUF + b
                gd.append(pltpu.async_copy(
                    cur.at[colv.at[ci]], msgs[b], gsem.at[b]))
            sd = []
            for b in range(NBUF):
                ci = blk * NBUF + b
                gd[b].wait()
                msg = msgs[b]
                for g in range(G // 16):
                    vv = valv[ci, pl.ds(g * 16, 16)]
                    for j in range(16):
                        e = g * 16 + j
                        bc = _bcast_lane(vv, j)
                        for q in range(H // 16):
                            sl = pl.ds(q * 16, 16)
                            msg[e, sl] = msg[e, sl] * bc
                sd.append(pltpu.async_copy(
                    msg, acc.at[rowv.at[ci]], ssem.at[b], add=True))
            for b in range(NBUF):
                sd[b].wait()
            return carry2

        lax.fori_loop(0, SB // NBUF, blk_body, 0)
        return carry

    lax.fori_loop(0, NSB, sb_body, 0)

    def tail_body(i, carry):
        crow = cb + NSB * SB + i
        pltpu.sync_copy(col2d.at[pl.ds(c * NCHUNKS + crow, 1)],
                        colv.at[pl.ds(0, 1)])
        pltpu.sync_copy(row2d.at[pl.ds(crow, 1)], rowv.at[pl.ds(0, 1)])
        pltpu.sync_copy(val2d.at[pl.ds(crow, 1)], valv.at[pl.ds(0, 1)])
        pltpu.sync_copy(cur.at[colv.at[0]], msg0)
        for g in range(G // 16):
            vv = valv[0, pl.ds(g * 16, 16)]
            for j in range(16):
                e = g * 16 + j
                bc = _bcast_lane(vv, j)
                for q in range(H // 16):
                    sl = pl.ds(q * 16, 16)
                    msg0[e, sl] = msg0[e, sl] * bc
        pltpu.sync_copy(msg0, acc.at[rowv.at[0]], add=True)
        return carry

    lax.fori_loop(0, nch - NSB * SB, tail_body, 0)
    plsc.subcore_barrier()
    o0 = pl.multiple_of(c * N + r0, 8)

    @pl.when(s < NS - 1)
    def _():
        pltpu.sync_copy(acc.at[pl.ds(r0, SA)], out.at[pl.ds(o0, SA)])

    @pl.when(s == NS - 1)
    def _():
        pltpu.sync_copy(acc.at[pl.ds((NS - 1) * SA, SA_LAST)],
                        out.at[pl.ds(pl.multiple_of(c * N + (NS - 1) * SA, 8),
                                     SA_LAST)])


@functools.partial(
    pl.kernel,
    out_type=(
        jax.ShapeDtypeStruct((3, NC, B, H), jnp.float32),
        jax.ShapeDtypeStruct((3, B, D), jnp.float32),
    ),
    mesh=_mesh,
    scratch_types=[
        pltpu.VMEM((1, G), jnp.int32),
        pltpu.VMEM((G, H), jnp.float32),
        pltpu.VMEM((G, H), jnp.float32),
        pltpu.VMEM((G, H), jnp.float32),
        pltpu.VMEM((G, H), jnp.float32),
        pltpu.VMEM((G, H), jnp.float32),
        pltpu.VMEM((G, D), jnp.float32),
        pltpu.VMEM((16,), jnp.float32),
    ],
    compiler_params=pltpu.CompilerParams(use_tc_tiling_on_sc=False, needs_layout_passes=False),
)
def _gather_combine(e0, e1, e2, e3, ego_full, idxcat, wpad, out_f, out_e,
                    idxv, b0, b1, b2, b3, obuf, ebuf, wbuf):
    c = lax.axis_index("c")
    s = lax.axis_index("s")
    pltpu.sync_copy(wpad, wbuf)
    wraw = wbuf[...]
    m = jnp.max(wraw)
    ex = jnp.exp(wraw - lax.broadcast(m, (16,)))
    ssum = jnp.sum(ex)
    wv = ex / lax.broadcast(ssum, (16,))
    bcw = [_bcast_lane(wv, t) for t in range(4)]

    per_sub = B // NS

    def jk_body(jk, carry):
        j = jk // 2
        k = jk - 2 * j
        off = s * per_sub + k * G
        src = pl.multiple_of((c * 3 + j) * B + off, G)
        pltpu.sync_copy(idxcat.at[pl.ds(src, G)], idxv.at[0])
        pltpu.sync_copy(e0.at[idxv.at[0]], b0)
        pltpu.sync_copy(e1.at[idxv.at[0]], b1)
        pltpu.sync_copy(e2.at[idxv.at[0]], b2)
        pltpu.sync_copy(e3.at[idxv.at[0]], b3)
        for e in range(G):
            for q in range(H // 16):
                sl = pl.ds(q * 16, 16)
                obuf[e, sl] = (b0[e, sl] * bcw[0] + b1[e, sl] * bcw[1]
                               + b2[e, sl] * bcw[2] + b3[e, sl] * bcw[3])
        pltpu.sync_copy(obuf, out_f.at[j, c, pl.ds(off, G)])
        return carry

    lax.fori_loop(0, 6, jk_body, 0)

    w = s * NC + c

    def ego_body(j, carry):
        pltpu.sync_copy(idxcat.at[pl.ds(pl.multiple_of(j * B + w * G, G), G)],
                        idxv.at[0])
        pltpu.sync_copy(ego_full.at[idxv.at[0]], ebuf)
        pltpu.sync_copy(ebuf, out_e.at[j, pl.ds(w * G, G)])
        return carry

    lax.fori_loop(0, 3, ego_body, 0)


def kernel(adj_indices, adj_values, users, pos_items, neg_items,
           user_table, item_table, layer_weights):
    ego_full = jnp.concatenate([user_table, item_table], axis=0)
    ego_flat = jnp.concatenate([ego_full[:, :H], ego_full[:, H:]], axis=0)
    row0 = adj_indices[0]
    col0 = adj_indices[1]
    row, col, valbits = lax.sort(
        (row0, col0, lax.bitcast_convert_type(adj_values, jnp.int32)),
        num_keys=1)
    adj_values = lax.bitcast_convert_type(valbits, jnp.float32)
    col2d = jnp.concatenate([col, col + N]).reshape(2 * NCHUNKS, G)
    row2d = row.reshape(NCHUNKS, G)
    val2d = adj_values.reshape(NCHUNKS, G)
    zeros = jnp.zeros((N, H), jnp.float32)
    wpad = jnp.concatenate(
        [layer_weights, jnp.full((12,), -1e30, jnp.float32)])
    iu = users
    ip = N_U + pos_items
    ineg = N_U + neg_items
    idxcat = jnp.concatenate(
        [iu, ip, ineg, iu + N, ip + N, ineg + N]).astype(jnp.int32)

    e1 = _prop(ego_flat, col2d, row2d, val2d, zeros)
    e2 = _prop(e1, col2d, row2d, val2d, zeros)
    e3 = _prop(e2, col2d, row2d, val2d, zeros)

    out_f, out_e = _gather_combine(ego_flat, e1, e2, e3, ego_full,
                                   idxcat, wpad)
    ue = jnp.concatenate([out_f[0, 0], out_f[0, 1]], axis=1)
    pe = jnp.concatenate([out_f[1, 0], out_f[1, 1]], axis=1)
    ne = jnp.concatenate([out_f[2, 0], out_f[2, 1]], axis=1)
    return ue, pe, ne, out_e[0], out_e[1], out_e[2]


# speedup vs baseline: 1.5151x; 1.5151x over previous
import functools

import jax
import jax.numpy as jnp
from jax import lax
from jax.experimental import pallas as pl
from jax.experimental.pallas import tpu as pltpu
from jax.experimental.pallas import tpu_sc as plsc

N_U = 12500
N_I = 37500
N = N_U + N_I
NN2 = 2 * N
D = 64
H = 32
E = 800000
B = 4096
NC = 2
NS = 16
G = 128
NCHUNKS = E // G
SA = 3128
SA_LAST = N - (NS - 1) * SA
NBUF = 4
SB = 16
NSB = 24
CH_FLOOR = NCHUNKS // NS
CH_EXTRA = NCHUNKS % NS

_mesh = plsc.VectorSubcoreMesh(core_axis_name="c", subcore_axis_name="s")

_GATHER_DNUMS = lax.GatherDimensionNumbers(
    offset_dims=(), collapsed_slice_dims=(0,), start_index_map=(0,))


def _bcast_lane(vec, j):
    idx = jnp.full((16, 1), j, jnp.int32)
    return lax.gather(vec, idx, _GATHER_DNUMS, (1,),
                      mode=lax.GatherScatterMode.PROMISE_IN_BOUNDS)


@functools.partial(
    pl.kernel,
    out_type=(
        jax.ShapeDtypeStruct((4 * NN2, H), jnp.float32),
        jax.ShapeDtypeStruct((3, NC, B, H), jnp.float32),
        jax.ShapeDtypeStruct((3, B, D), jnp.float32),
    ),
    mesh=_mesh,
    scratch_types=[
        pltpu.VMEM_SHARED((N, H), jnp.float32),
        pltpu.VMEM((SB, G), jnp.int32),
        pltpu.VMEM((SB, G), jnp.int32),
        pltpu.VMEM((SB, G), jnp.float32),
        pltpu.VMEM((G, H), jnp.float32),
        pltpu.VMEM((G, H), jnp.float32),
        pltpu.VMEM((G, H), jnp.float32),
        pltpu.VMEM((G, H), jnp.float32),
        pltpu.VMEM((G // 2, D), jnp.float32),
        pltpu.VMEM((16,), jnp.float32),
        pltpu.SemaphoreType.DMA((NBUF,)),
        pltpu.SemaphoreType.DMA((NBUF,)),
    ],
    compiler_params=pltpu.CompilerParams(use_tc_tiling_on_sc=False,
                                         needs_layout_passes=False),
)
def _lightgcn(ego_flat, col2d, row2d, val2d, zeros, ego_full, idxcat, wpad,
              eall, out_f, out_e,
              acc, colv, rowv, valv, msg0, msg1, msg2, msg3, ebuf, wbuf,
              gsem, ssem):
    c = lax.axis_index("c")
    s = lax.axis_index("s")
    msgs = (msg0, msg1, msg2, msg3)
    r0 = pl.multiple_of(s * SA, 8)
    o0 = pl.multiple_of(c * N + r0, 8)
    o0_last = pl.multiple_of(c * N + (NS - 1) * SA, 8)

    @pl.when(s < NS - 1)
    def _():
        pltpu.sync_copy(ego_flat.at[pl.ds(o0, SA)], eall.at[pl.ds(o0, SA)])

    @pl.when(s == NS - 1)
    def _():
        pltpu.sync_copy(ego_flat.at[pl.ds(o0_last, SA_LAST)],
                        eall.at[pl.ds(o0_last, SA_LAST)])

    cb = s * CH_FLOOR + jnp.minimum(s, CH_EXTRA)
    nch = jnp.where(s < CH_EXTRA, CH_FLOOR + 1, CH_FLOOR)

    def layer(k, carry):
        @pl.when(s < NS - 1)
        def _():
            pltpu.sync_copy(zeros.at[pl.ds(r0, SA)], acc.at[pl.ds(r0, SA)])

        @pl.when(s == NS - 1)
        def _():
            pltpu.sync_copy(zeros.at[pl.ds((NS - 1) * SA, SA_LAST)],
                            acc.at[pl.ds((NS - 1) * SA, SA_LAST)])

        plsc.subcore_barrier()
        src = eall.at[pl.ds(pl.multiple_of(k * NN2, 8), NN2)]

        def sb_body(sb, carry1):
            crow = cb + sb * SB
            pltpu.sync_copy(col2d.at[pl.ds(c * NCHUNKS + crow, SB)], colv)
            pltpu.sync_copy(row2d.at[pl.ds(crow, SB)], rowv)
            pltpu.sync_copy(val2d.at[pl.ds(crow, SB)], valv)

            def blk_body(blk, carry2):
                gd = []
                for b in range(NBUF):
                    ci = blk * NBUF + b
                    gd.append(pltpu.async_copy(
                        src.at[colv.at[ci]], msgs[b], gsem.at[b]))
                sd = []
                for b in range(NBUF):
                    ci = blk * NBUF + b
                    gd[b].wait()
                    msg = msgs[b]

                    def scale_grp(g, cc, msg=msg, ci=ci):
                        vv = valv[ci, pl.ds(g * 16, 16)]
                        for j in range(16):
                            e = g * 16 + j
                            bc = _bcast_lane(vv, j)
                            for q in range(H // 16):
                                sl = pl.ds(q * 16, 16)
                                msg[e, sl] = msg[e, sl] * bc
                        return cc

                    lax.fori_loop(0, G // 16, scale_grp, 0)
                    sd.append(pltpu.async_copy(
                        msg, acc.at[rowv.at[ci]], ssem.at[b], add=True))
                for b in range(NBUF):
                    sd[b].wait()
                return carry2

            lax.fori_loop(0, SB // NBUF, blk_body, 0)
            return carry1

        lax.fori_loop(0, NSB, sb_body, 0)

        def tail_body(i, carry1):
            crow = cb + NSB * SB + i
            pltpu.sync_copy(col2d.at[pl.ds(c * NCHUNKS + crow, 1)],
                            colv.at[pl.ds(0, 1)])
            pltpu.sync_copy(row2d.at[pl.ds(crow, 1)], rowv.at[pl.ds(0, 1)])
            pltpu.sync_copy(val2d.at[pl.ds(crow, 1)], valv.at[pl.ds(0, 1)])
            pltpu.sync_copy(src.at[colv.at[0]], msg0)

            def tail_grp(g, cc):
                vv = valv[0, pl.ds(g * 16, 16)]
                for j in range(16):
                    e = g * 16 + j
                    bc = _bcast_lane(vv, j)
                    for q in range(H // 16):
                        sl = pl.ds(q * 16, 16)
                        msg0[e, sl] = msg0[e, sl] * bc
                return cc

            lax.fori_loop(0, G // 16, tail_grp, 0)
            pltpu.sync_copy(msg0, acc.at[rowv.at[0]], add=True)
            return carry1

        lax.fori_loop(0, nch - NSB * SB, tail_body, 0)
        plsc.subcore_barrier()

        d0 = pl.multiple_of((k + 1) * NN2 + o0, 8)
        d0_last = pl.multiple_of((k + 1) * NN2 + o0_last, 8)

        @pl.when(s < NS - 1)
        def _():
            pltpu.sync_copy(acc.at[pl.ds(r0, SA)], eall.at[pl.ds(d0, SA)])

        @pl.when(s == NS - 1)
        def _():
            pltpu.sync_copy(acc.at[pl.ds((NS - 1) * SA, SA_LAST)],
                            eall.at[pl.ds(d0_last, SA_LAST)])

        plsc.subcore_barrier()
        return carry

    lax.fori_loop(0, 3, layer, 0)

    pltpu.sync_copy(wpad, wbuf)
    wraw = wbuf[...]
    m = jnp.max(wraw)
    ex = jnp.exp(wraw - lax.broadcast(m, (16,)))
    ssum = jnp.sum(ex)
    wv = ex / lax.broadcast(ssum, (16,))
    bcw = [_bcast_lane(wv, t) for t in range(4)]
    secs = [eall.at[pl.ds(k * NN2, NN2)] for k in range(4)]

    per_sub = B // NS

    def jk_body(jk, carry):
        j = jk // 2
        k = jk - 2 * j
        off = s * per_sub + k * G
        isrc = pl.multiple_of((c * 3 + j) * B + off, G)
        pltpu.sync_copy(idxcat.at[pl.ds(isrc, G)], colv.at[0])
        for t in range(4):
            pltpu.sync_copy(secs[t].at[colv.at[0]], msgs[t])
        def comb_row(e, cc):
            for q in range(H // 16):
                sl = pl.ds(q * 16, 16)
                msg0[e, sl] = (msg0[e, sl] * bcw[0] + msg1[e, sl] * bcw[1]
                               + msg2[e, sl] * bcw[2] + msg3[e, sl] * bcw[3])
            return cc

        lax.fori_loop(0, G, comb_row, 0)
        pltpu.sync_copy(msg0, out_f.at[j, c, pl.ds(off, G)])
        return carry

    lax.fori_loop(0, 6, jk_body, 0)

    w = s * NC + c

    def ego_body(j, carry):
        pltpu.sync_copy(idxcat.at[pl.ds(pl.multiple_of(j * B + w * G, G), G)],
                        colv.at[0])
        for half in range(2):
            pltpu.sync_copy(
                ego_full.at[colv.at[0, pl.ds(half * (G // 2), G // 2)]], ebuf)
            pltpu.sync_copy(
                ebuf, out_e.at[j, pl.ds(w * G + half * (G // 2), G // 2)])
        return carry

    lax.fori_loop(0, 3, ego_body, 0)


def kernel(adj_indices, adj_values, users, pos_items, neg_items,
           user_table, item_table, layer_weights):
    ego_full = jnp.concatenate([user_table, item_table], axis=0)
    ego_flat = jnp.concatenate([ego_full[:, :H], ego_full[:, H:]], axis=0)
    row = adj_indices[0]
    col = adj_indices[1]
    col2d = jnp.concatenate([col, col + N]).reshape(2 * NCHUNKS, G)
    row2d = row.reshape(NCHUNKS, G)
    val2d = adj_values.reshape(NCHUNKS, G)
    zeros = jnp.zeros((N, H), jnp.float32)
    wpad = jnp.concatenate(
        [layer_weights, jnp.full((12,), -1e30, jnp.float32)])
    iu = users
    ip = N_U + pos_items
    ineg = N_U + neg_items
    idxcat = jnp.concatenate(
        [iu, ip, ineg, iu + N, ip + N, ineg + N]).astype(jnp.int32)

    _, out_f, out_e = _lightgcn(ego_flat, col2d, row2d, val2d, zeros,
                                ego_full, idxcat, wpad)
    ue = jnp.concatenate([out_f[0, 0], out_f[0, 1]], axis=1)
    pe = jnp.concatenate([out_f[1, 0], out_f[1, 1]], axis=1)
    ne = jnp.concatenate([out_f[2, 0], out_f[2, 1]], axis=1)
    return ue, pe, ne, out_e[0], out_e[1], out_e[2]


# speedup vs baseline: 1.5168x; 1.0011x over previous
import functools

import jax
import jax.numpy as jnp
from jax import lax
from jax.experimental import pallas as pl
from jax.experimental.pallas import tpu as pltpu
from jax.experimental.pallas import tpu_sc as plsc

N_U = 12500
N_I = 37500
N = N_U + N_I
NN2 = 2 * N
D = 64
H = 32
E = 800000
B = 4096
NC = 2
NS = 16
G = 128
NCHUNKS = E // G
SA = 3128
SA_LAST = N - (NS - 1) * SA
NBUF = 4
SB = 16
NSB = 24
CH_FLOOR = NCHUNKS // NS
CH_EXTRA = NCHUNKS % NS

_mesh = plsc.VectorSubcoreMesh(core_axis_name="c", subcore_axis_name="s")

_GATHER_DNUMS = lax.GatherDimensionNumbers(
    offset_dims=(), collapsed_slice_dims=(0,), start_index_map=(0,))


def _bcast_lane(vec, j):
    idx = jnp.full((16, 1), j, jnp.int32)
    return lax.gather(vec, idx, _GATHER_DNUMS, (1,),
                      mode=lax.GatherScatterMode.PROMISE_IN_BOUNDS)


@functools.partial(
    pl.kernel,
    out_type=(
        jax.ShapeDtypeStruct((4 * NN2, H), jnp.float32),
        jax.ShapeDtypeStruct((3, NC, B, H), jnp.float32),
        jax.ShapeDtypeStruct((3, B, D), jnp.float32),
    ),
    mesh=_mesh,
    scratch_types=[
        pltpu.VMEM_SHARED((N, H), jnp.float32),
        pltpu.VMEM((SB, G), jnp.int32),
        pltpu.VMEM((SB, G), jnp.int32),
        pltpu.VMEM((SB, G), jnp.float32),
        pltpu.VMEM((G, H), jnp.float32),
        pltpu.VMEM((G, H), jnp.float32),
        pltpu.VMEM((G, H), jnp.float32),
        pltpu.VMEM((G, H), jnp.float32),
        pltpu.VMEM((G // 2, D), jnp.float32),
        pltpu.VMEM((16,), jnp.float32),
        pltpu.SemaphoreType.DMA((NBUF,)),
        pltpu.SemaphoreType.DMA((NBUF,)),
    ],
    compiler_params=pltpu.CompilerParams(use_tc_tiling_on_sc=False,
                                         needs_layout_passes=False),
)
def _lightgcn(ego_flat, col2d, row2d, val2d, zeros, ego_full, idxcat, wpad,
              eall, out_f, out_e,
              acc, colv, rowv, valv, msg0, msg1, msg2, msg3, ebuf, wbuf,
              gsem, ssem):
    c = lax.axis_index("c")
    s = lax.axis_index("s")
    msgs = (msg0, msg1, msg2, msg3)
    r0 = pl.multiple_of(s * SA, 8)
    o0 = pl.multiple_of(c * N + r0, 8)
    o0_last = pl.multiple_of(c * N + (NS - 1) * SA, 8)

    @pl.when(s < NS - 1)
    def _():
        pltpu.sync_copy(ego_flat.at[pl.ds(o0, SA)], eall.at[pl.ds(o0, SA)])

    @pl.when(s == NS - 1)
    def _():
        pltpu.sync_copy(ego_flat.at[pl.ds(o0_last, SA_LAST)],
                        eall.at[pl.ds(o0_last, SA_LAST)])

    cb = s * CH_FLOOR + jnp.minimum(s, CH_EXTRA)
    nch = jnp.where(s < CH_EXTRA, CH_FLOOR + 1, CH_FLOOR)

    def layer(k, carry):
        @pl.when(s < NS - 1)
        def _():
            pltpu.sync_copy(zeros.at[pl.ds(r0, SA)], acc.at[pl.ds(r0, SA)])

        @pl.when(s == NS - 1)
        def _():
            pltpu.sync_copy(zeros.at[pl.ds((NS - 1) * SA, SA_LAST)],
                            acc.at[pl.ds((NS - 1) * SA, SA_LAST)])

        plsc.subcore_barrier()
        src = eall.at[pl.ds(pl.multiple_of(k * NN2, 8), NN2)]

        def sb_body(sb, carry1):
            crow = cb + sb * SB
            pltpu.sync_copy(col2d.at[pl.ds(c * NCHUNKS + crow, SB)], colv)
            pltpu.sync_copy(row2d.at[pl.ds(crow, SB)], rowv)
            pltpu.sync_copy(val2d.at[pl.ds(crow, SB)], valv)

            def blk_body(blk, carry2):
                gd = []
                for b in range(NBUF):
                    ci = blk * NBUF + b
                    gd.append(pltpu.async_copy(
                        src.at[colv.at[ci]], msgs[b], gsem.at[b]))
                sd = []
                for b in range(NBUF):
                    ci = blk * NBUF + b
                    gd[b].wait()
                    msg = msgs[b]
                    for g in range(G // 16):
                        vv = valv[ci, pl.ds(g * 16, 16)]
                        for j in range(16):
                            e = g * 16 + j
                            bc = _bcast_lane(vv, j)
                            for q in range(H // 16):
                                sl = pl.ds(q * 16, 16)
                                msg[e, sl] = msg[e, sl] * bc
                    sd.append(pltpu.async_copy(
                        msg, acc.at[rowv.at[ci]], ssem.at[b], add=True))
                for b in range(NBUF):
                    sd[b].wait()
                return carry2

            lax.fori_loop(0, SB // NBUF, blk_body, 0)
            return carry1

        lax.fori_loop(0, NSB, sb_body, 0)

        def tail_body(i, carry1):
            crow = cb + NSB * SB + i
            pltpu.sync_copy(col2d.at[pl.ds(c * NCHUNKS + crow, 1)],
                            colv.at[pl.ds(0, 1)])
            pltpu.sync_copy(row2d.at[pl.ds(crow, 1)], rowv.at[pl.ds(0, 1)])
            pltpu.sync_copy(val2d.at[pl.ds(crow, 1)], valv.at[pl.ds(0, 1)])
            pltpu.sync_copy(src.at[colv.at[0]], msg0)

            def tail_grp(g, cc):
                vv = valv[0, pl.ds(g * 16, 16)]
                for j in range(16):
                    e = g * 16 + j
                    bc = _bcast_lane(vv, j)
                    for q in range(H // 16):
                        sl = pl.ds(q * 16, 16)
                        msg0[e, sl] = msg0[e, sl] * bc
                return cc

            lax.fori_loop(0, G // 16, tail_grp, 0)
            pltpu.sync_copy(msg0, acc.at[rowv.at[0]], add=True)
            return carry1

        lax.fori_loop(0, nch - NSB * SB, tail_body, 0)
        plsc.subcore_barrier()

        d0 = pl.multiple_of((k + 1) * NN2 + o0, 8)
        d0_last = pl.multiple_of((k + 1) * NN2 + o0_last, 8)

        @pl.when(s < NS - 1)
        def _():
            pltpu.sync_copy(acc.at[pl.ds(r0, SA)], eall.at[pl.ds(d0, SA)])

        @pl.when(s == NS - 1)
        def _():
            pltpu.sync_copy(acc.at[pl.ds((NS - 1) * SA, SA_LAST)],
                            eall.at[pl.ds(d0_last, SA_LAST)])

        plsc.subcore_barrier()
        return carry

    lax.fori_loop(0, 3, layer, 0)

    pltpu.sync_copy(wpad, wbuf)
    wraw = wbuf[...]
    m = jnp.max(wraw)
    ex = jnp.exp(wraw - lax.broadcast(m, (16,)))
    ssum = jnp.sum(ex)
    wv = ex / lax.broadcast(ssum, (16,))
    bcw = [_bcast_lane(wv, t) for t in range(4)]
    secs = [eall.at[pl.ds(k * NN2, NN2)] for k in range(4)]

    per_sub = B // NS

    def jk_body(jk, carry):
        j = jk // 2
        k = jk - 2 * j
        off = s * per_sub + k * G
        isrc = pl.multiple_of((c * 3 + j) * B + off, G)
        pltpu.sync_copy(idxcat.at[pl.ds(isrc, G)], colv.at[0])
        for t in range(4):
            pltpu.sync_copy(secs[t].at[colv.at[0]], msgs[t])
        def comb_row(e, cc):
            for q in range(H // 16):
                sl = pl.ds(q * 16, 16)
                msg0[e, sl] = (msg0[e, sl] * bcw[0] + msg1[e, sl] * bcw[1]
                               + msg2[e, sl] * bcw[2] + msg3[e, sl] * bcw[3])
            return cc

        lax.fori_loop(0, G, comb_row, 0)
        pltpu.sync_copy(msg0, out_f.at[j, c, pl.ds(off, G)])
        return carry

    lax.fori_loop(0, 6, jk_body, 0)

    w = s * NC + c

    def ego_body(j, carry):
        pltpu.sync_copy(idxcat.at[pl.ds(pl.multiple_of(j * B + w * G, G), G)],
                        colv.at[0])
        for half in range(2):
            pltpu.sync_copy(
                ego_full.at[colv.at[0, pl.ds(half * (G // 2), G // 2)]], ebuf)
            pltpu.sync_copy(
                ebuf, out_e.at[j, pl.ds(w * G + half * (G // 2), G // 2)])
        return carry

    lax.fori_loop(0, 3, ego_body, 0)


def kernel(adj_indices, adj_values, users, pos_items, neg_items,
           user_table, item_table, layer_weights):
    ego_full = jnp.concatenate([user_table, item_table], axis=0)
    ego_flat = jnp.concatenate([ego_full[:, :H], ego_full[:, H:]], axis=0)
    row = adj_indices[0]
    col = adj_indices[1]
    col2d = jnp.concatenate([col, col + N]).reshape(2 * NCHUNKS, G)
    row2d = row.reshape(NCHUNKS, G)
    val2d = adj_values.reshape(NCHUNKS, G)
    zeros = jnp.zeros((N, H), jnp.float32)
    wpad = jnp.concatenate(
        [layer_weights, jnp.full((12,), -1e30, jnp.float32)])
    iu = users
    ip = N_U + pos_items
    ineg = N_U + neg_items
    idxcat = jnp.concatenate(
        [iu, ip, ineg, iu + N, ip + N, ineg + N]).astype(jnp.int32)

    _, out_f, out_e = _lightgcn(ego_flat, col2d, row2d, val2d, zeros,
                                ego_full, idxcat, wpad)
    ue = jnp.concatenate([out_f[0, 0], out_f[0, 1]], axis=1)
    pe = jnp.concatenate([out_f[1, 0], out_f[1, 1]], axis=1)
    ne = jnp.concatenate([out_f[2, 0], out_f[2, 1]], axis=1)
    return ue, pe, ne, out_e[0], out_e[1], out_e[2]


# speedup vs baseline: 1.6240x; 1.0707x over previous
import functools

import jax
import jax.numpy as jnp
from jax import lax
from jax.experimental import pallas as pl
from jax.experimental.pallas import tpu as pltpu
from jax.experimental.pallas import tpu_sc as plsc

N_U = 12500
N_I = 37500
N = N_U + N_I
NN2 = 2 * N
D = 64
H = 32
E = 800000
B = 4096
NC = 2
NS = 16
G = 128
NCHUNKS = E // G
SA = 3128
SA_LAST = N - (NS - 1) * SA
NBUF = 4
SB = 32
NSB = 12
CH_FLOOR = NCHUNKS // NS
CH_EXTRA = NCHUNKS % NS

_mesh = plsc.VectorSubcoreMesh(core_axis_name="c", subcore_axis_name="s")

_GATHER_DNUMS = lax.GatherDimensionNumbers(
    offset_dims=(), collapsed_slice_dims=(0,), start_index_map=(0,))


def _bcast_lane(vec, j):
    idx = jnp.full((16, 1), j, jnp.int32)
    return lax.gather(vec, idx, _GATHER_DNUMS, (1,),
                      mode=lax.GatherScatterMode.PROMISE_IN_BOUNDS)


@functools.partial(
    pl.kernel,
    out_type=jax.ShapeDtypeStruct((4 * NN2, H), jnp.float32),
    mesh=_mesh,
    scratch_types=[
        pltpu.VMEM_SHARED((N, H), jnp.float32),
        pltpu.VMEM((SB, G), jnp.int32),
        pltpu.VMEM((SB, G), jnp.int32),
        pltpu.VMEM((SB, G), jnp.float32),
        pltpu.VMEM((G, H), jnp.float32),
        pltpu.VMEM((G, H), jnp.float32),
        pltpu.VMEM((G, H), jnp.float32),
        pltpu.VMEM((G, H), jnp.float32),
        pltpu.SemaphoreType.DMA((NBUF,)),
        pltpu.SemaphoreType.DMA((NBUF,)),
    ],
    compiler_params=pltpu.CompilerParams(use_tc_tiling_on_sc=False,
                                         needs_layout_passes=False),
)
def _prop3(ego_flat, col2d, row2d, val2d, zeros, eall,
           acc, colv, rowv, valv, msg0, msg1, msg2, msg3, gsem, ssem):
    c = lax.axis_index("c")
    s = lax.axis_index("s")
    msgs = (msg0, msg1, msg2, msg3)
    r0 = pl.multiple_of(s * SA, 8)
    o0 = pl.multiple_of(c * N + r0, 8)
    o0_last = pl.multiple_of(c * N + (NS - 1) * SA, 8)

    @pl.when(s < NS - 1)
    def _():
        pltpu.sync_copy(ego_flat.at[pl.ds(o0, SA)], eall.at[pl.ds(o0, SA)])

    @pl.when(s == NS - 1)
    def _():
        pltpu.sync_copy(ego_flat.at[pl.ds(o0_last, SA_LAST)],
                        eall.at[pl.ds(o0_last, SA_LAST)])

    cb = s * CH_FLOOR + jnp.minimum(s, CH_EXTRA)
    nch = jnp.where(s < CH_EXTRA, CH_FLOOR + 1, CH_FLOOR)

    def layer(k, carry):
        @pl.when(s < NS - 1)
        def _():
            pltpu.sync_copy(zeros.at[pl.ds(r0, SA)], acc.at[pl.ds(r0, SA)])

        @pl.when(s == NS - 1)
        def _():
            pltpu.sync_copy(zeros.at[pl.ds((NS - 1) * SA, SA_LAST)],
                            acc.at[pl.ds((NS - 1) * SA, SA_LAST)])

        plsc.subcore_barrier()
        src = eall.at[pl.ds(pl.multiple_of(k * NN2, 8), NN2)]

        def sb_body(sb, carry1):
            crow = cb + sb * SB
            pltpu.sync_copy(col2d.at[pl.ds(c * NCHUNKS + crow, SB)], colv)
            pltpu.sync_copy(row2d.at[pl.ds(crow, SB)], rowv)
            pltpu.sync_copy(val2d.at[pl.ds(crow, SB)], valv)

            def blk_body(blk, carry2):
                gd = []
                for b in range(NBUF):
                    ci = blk * NBUF + b
                    gd.append(pltpu.async_copy(
                        src.at[colv.at[ci]], msgs[b], gsem.at[b]))
                sd = []
                for b in range(NBUF):
                    ci = blk * NBUF + b
                    gd[b].wait()
                    msg = msgs[b]
                    for g in range(G // 16):
                        vv = valv[ci, pl.ds(g * 16, 16)]
                        for j in range(16):
                            e = g * 16 + j
                            bc = _bcast_lane(vv, j)
                            for q in range(H // 16):
                                sl = pl.ds(q * 16, 16)
                                msg[e, sl] = msg[e, sl] * bc
                    sd.append(pltpu.async_copy(
                        msg, acc.at[rowv.at[ci]], ssem.at[b], add=True))
                for b in range(NBUF):
                    sd[b].wait()
                return carry2

            lax.fori_loop(0, SB // NBUF, blk_body, 0)
            return carry1

        lax.fori_loop(0, NSB, sb_body, 0)

        def tail_body(i, carry1):
            crow = cb + NSB * SB + i
            pltpu.sync_copy(col2d.at[pl.ds(c * NCHUNKS + crow, 1)],
                            colv.at[pl.ds(0, 1)])
            pltpu.sync_copy(row2d.at[pl.ds(crow, 1)], rowv.at[pl.ds(0, 1)])
            pltpu.sync_copy(val2d.at[pl.ds(crow, 1)], valv.at[pl.ds(0, 1)])
            pltpu.sync_copy(src.at[colv.at[0]], msg0)

            def tail_grp(g, cc):
                vv = valv[0, pl.ds(g * 16, 16)]
                for j in range(16):
                    e = g * 16 + j
                    bc = _bcast_lane(vv, j)
                    for q in range(H // 16):
                        sl = pl.ds(q * 16, 16)
                        msg0[e, sl] = msg0[e, sl] * bc
                return cc

            lax.fori_loop(0, G // 16, tail_grp, 0)
            pltpu.sync_copy(msg0, acc.at[rowv.at[0]], add=True)
            return carry1

        lax.fori_loop(0, nch - NSB * SB, tail_body, 0)
        plsc.subcore_barrier()

        d0 = pl.multiple_of((k + 1) * NN2 + o0, 8)
        d0_last = pl.multiple_of((k + 1) * NN2 + o0_last, 8)

        @pl.when(s < NS - 1)
        def _():
            pltpu.sync_copy(acc.at[pl.ds(r0, SA)], eall.at[pl.ds(d0, SA)])

        @pl.when(s == NS - 1)
        def _():
            pltpu.sync_copy(acc.at[pl.ds((NS - 1) * SA, SA_LAST)],
                            eall.at[pl.ds(d0_last, SA_LAST)])

        plsc.subcore_barrier()
        return carry

    lax.fori_loop(0, 3, layer, 0)


@functools.partial(
    pl.kernel,
    out_type=(
        jax.ShapeDtypeStruct((3, NC, B, H), jnp.float32),
        jax.ShapeDtypeStruct((3, B, D), jnp.float32),
    ),
    mesh=_mesh,
    scratch_types=[
        pltpu.VMEM((1, G), jnp.int32),
        pltpu.VMEM((G, H), jnp.float32),
        pltpu.VMEM((G, H), jnp.float32),
        pltpu.VMEM((G, H), jnp.float32),
        pltpu.VMEM((G, H), jnp.float32),
        pltpu.VMEM((G, H), jnp.float32),
        pltpu.VMEM((G, D), jnp.float32),
        pltpu.VMEM((16,), jnp.float32),
    ],
    compiler_params=pltpu.CompilerParams(use_tc_tiling_on_sc=False,
                                         needs_layout_passes=False),
)
def _gather_combine(eall, ego_full, idxcat, wpad, out_f, out_e,
                    idxv, b0, b1, b2, b3, obuf, ebuf, wbuf):
    c = lax.axis_index("c")
    s = lax.axis_index("s")
    bufs = (b0, b1, b2, b3)
    pltpu.sync_copy(wpad, wbuf)
    wraw = wbuf[...]
    m = jnp.max(wraw)
    ex = jnp.exp(wraw - lax.broadcast(m, (16,)))
    ssum = jnp.sum(ex)
    wv = ex / lax.broadcast(ssum, (16,))
    bcw = [_bcast_lane(wv, t) for t in range(4)]
    secs = [eall.at[pl.ds(k * NN2, NN2)] for k in range(4)]

    per_sub = B // NS

    def jk_body(jk, carry):
        j = jk // 2
        k = jk - 2 * j
        off = s * per_sub + k * G
        isrc = pl.multiple_of((c * 3 + j) * B + off, G)
        pltpu.sync_copy(idxcat.at[pl.ds(isrc, G)], idxv.at[0])
        for t in range(4):
            pltpu.sync_copy(secs[t].at[idxv.at[0]], bufs[t])
        for e in range(G):
            for q in range(H // 16):
                sl = pl.ds(q * 16, 16)
                obuf[e, sl] = (b0[e, sl] * bcw[0] + b1[e, sl] * bcw[1]
                               + b2[e, sl] * bcw[2] + b3[e, sl] * bcw[3])
        pltpu.sync_copy(obuf, out_f.at[j, c, pl.ds(off, G)])
        return carry

    lax.fori_loop(0, 6, jk_body, 0)

    w = s * NC + c

    def ego_body(j, carry):
        pltpu.sync_copy(idxcat.at[pl.ds(pl.multiple_of(j * B + w * G, G), G)],
                        idxv.at[0])
        pltpu.sync_copy(ego_full.at[idxv.at[0]], ebuf)
        pltpu.sync_copy(ebuf, out_e.at[j, pl.ds(w * G, G)])
        return carry

    lax.fori_loop(0, 3, ego_body, 0)


def kernel(adj_indices, adj_values, users, pos_items, neg_items,
           user_table, item_table, layer_weights):
    ego_full = jnp.concatenate([user_table, item_table], axis=0)
    ego_flat = jnp.concatenate([ego_full[:, :H], ego_full[:, H:]], axis=0)
    row = adj_indices[0]
    col = adj_indices[1]
    col2d = jnp.concatenate([col, col + N]).reshape(2 * NCHUNKS, G)
    row2d = row.reshape(NCHUNKS, G)
    val2d = adj_values.reshape(NCHUNKS, G)
    zeros = jnp.zeros((N, H), jnp.float32)
    wpad = jnp.concatenate(
        [layer_weights, jnp.full((12,), -1e30, jnp.float32)])
    iu = users
    ip = N_U + pos_items
    ineg = N_U + neg_items
    idxcat = jnp.concatenate(
        [iu, ip, ineg, iu + N, ip + N, ineg + N]).astype(jnp.int32)

    eall = _prop3(ego_flat, col2d, row2d, val2d, zeros)
    out_f, out_e = _gather_combine(eall, ego_full, idxcat, wpad)
    ue = jnp.concatenate([out_f[0, 0], out_f[0, 1]], axis=1)
    pe = jnp.concatenate([out_f[1, 0], out_f[1, 1]], axis=1)
    ne = jnp.concatenate([out_f[2, 0], out_f[2, 1]], axis=1)
    return ue, pe, ne, out_e[0], out_e[1], out_e[2]


# speedup vs baseline: 2.3220x; 1.4298x over previous
import functools

import jax
import jax.numpy as jnp
from jax import lax
from jax.experimental import pallas as pl
from jax.experimental.pallas import tpu as pltpu
from jax.experimental.pallas import tpu_sc as plsc

N_U = 12500
N_I = 37500
N = N_U + N_I
D = 64
H = 32
E = 800000
B = 4096
NC = 2
NS = 16
G = 128
NCHUNKS = E // G
SA = 3128
SA_LAST = N - (NS - 1) * SA
NBUF = 4
SB = 32
NSB = 12
CH_FLOOR = NCHUNKS // NS
CH_EXTRA = NCHUNKS % NS

_mesh = plsc.VectorSubcoreMesh(core_axis_name="c", subcore_axis_name="s")

_GATHER_DNUMS = lax.GatherDimensionNumbers(
    offset_dims=(), collapsed_slice_dims=(0,), start_index_map=(0,))


def _bcast_lane(vec, j):
    idx = jnp.full((16, 1), j, jnp.int32)
    return lax.gather(vec, idx, _GATHER_DNUMS, (1,),
                      mode=lax.GatherScatterMode.PROMISE_IN_BOUNDS)


@functools.partial(
    pl.kernel,
    out_type=jax.ShapeDtypeStruct((2 * N, H), jnp.float32),
    mesh=_mesh,
    scratch_types=[
        pltpu.VMEM_SHARED((N, H), jnp.float32),
        pltpu.VMEM((SB, G), jnp.int32),
        pltpu.VMEM((SB, G), jnp.int32),
        pltpu.VMEM((SB, G), jnp.float32),
        pltpu.VMEM((G, H), jnp.float32),
        pltpu.VMEM((G, H), jnp.float32),
        pltpu.VMEM((G, H), jnp.float32),
        pltpu.VMEM((G, H), jnp.float32),
        pltpu.SemaphoreType.DMA((NBUF,)),
        pltpu.SemaphoreType.DMA((NBUF,)),
    ],
    compiler_params=pltpu.CompilerParams(use_tc_tiling_on_sc=False, needs_layout_passes=False),
)
def _prop(cur, col2d, row2d, val2d, zeros, out, acc, colv, rowv, valv,
          msg0, msg1, msg2, msg3, gsem, ssem):
    c = lax.axis_index("c")
    s = lax.axis_index("s")
    msgs = (msg0, msg1, msg2, msg3)
    r0 = pl.multiple_of(s * SA, 8)

    @pl.when(s < NS - 1)
    def _():
        pltpu.sync_copy(zeros.at[pl.ds(r0, SA)], acc.at[pl.ds(r0, SA)])

    @pl.when(s == NS - 1)
    def _():
        pltpu.sync_copy(zeros.at[pl.ds((NS - 1) * SA, SA_LAST)],
                        acc.at[pl.ds((NS - 1) * SA, SA_LAST)])

    plsc.subcore_barrier()

    cb = s * CH_FLOOR + jnp.minimum(s, CH_EXTRA)
    nch = jnp.where(s < CH_EXTRA, CH_FLOOR + 1, CH_FLOOR)

    def sb_body(sb, carry):
        crow = cb + sb * SB
        pltpu.sync_copy(col2d.at[pl.ds(c * NCHUNKS + crow, SB)], colv)
        pltpu.sync_copy(row2d.at[pl.ds(crow, SB)], rowv)
        pltpu.sync_copy(val2d.at[pl.ds(crow, SB)], valv)

        def blk_body(blk, carry2, sb=sb):
            gblk = sb * (SB // NBUF) + blk

            for b in range(NBUF):
                ci = blk * NBUF + b

                @pl.when(gblk > 0)
                def _(b=b):
                    pltpu.make_async_copy(
                        zeros.at[pl.ds(0, G)], msgs[b], ssem.at[b]).wait()

                pltpu.async_copy(cur.at[colv.at[ci]], msgs[b], gsem.at[b])
            for b in range(NBUF):
                ci = blk * NBUF + b
                pltpu.make_async_copy(
                    cur.at[colv.at[ci]], msgs[b], gsem.at[b]).wait()
                msg = msgs[b]
                for g in range(G // 16):
                    vv = valv[ci, pl.ds(g * 16, 16)]
                    for j in range(16):
                        e = g * 16 + j
                        bc = _bcast_lane(vv, j)
                        for q in range(H // 16):
                            sl = pl.ds(q * 16, 16)
                            msg[e, sl] = msg[e, sl] * bc
                pltpu.async_copy(
                    msg, acc.at[rowv.at[ci]], ssem.at[b], add=True)
            return carry2

        lax.fori_loop(0, SB // NBUF, blk_body, 0)
        return carry

    lax.fori_loop(0, NSB, sb_body, 0)
    for b in range(NBUF):
        pltpu.make_async_copy(zeros.at[pl.ds(0, G)], msgs[b], ssem.at[b]).wait()

    def tail_body(i, carry):
        crow = cb + NSB * SB + i
        pltpu.sync_copy(col2d.at[pl.ds(c * NCHUNKS + crow, 1)],
                        colv.at[pl.ds(0, 1)])
        pltpu.sync_copy(row2d.at[pl.ds(crow, 1)], rowv.at[pl.ds(0, 1)])
        pltpu.sync_copy(val2d.at[pl.ds(crow, 1)], valv.at[pl.ds(0, 1)])
        pltpu.sync_copy(cur.at[colv.at[0]], msg0)
        for g in range(G // 16):
            vv = valv[0, pl.ds(g * 16, 16)]
            for j in range(16):
                e = g * 16 + j
                bc = _bcast_lane(vv, j)
                for q in range(H // 16):
                    sl = pl.ds(q * 16, 16)
                    msg0[e, sl] = msg0[e, sl] * bc
        pltpu.sync_copy(msg0, acc.at[rowv.at[0]], add=True)
        return carry

    lax.fori_loop(0, nch - NSB * SB, tail_body, 0)
    plsc.subcore_barrier()
    o0 = pl.multiple_of(c * N + r0, 8)

    @pl.when(s < NS - 1)
    def _():
        pltpu.sync_copy(acc.at[pl.ds(r0, SA)], out.at[pl.ds(o0, SA)])

    @pl.when(s == NS - 1)
    def _():
        pltpu.sync_copy(acc.at[pl.ds((NS - 1) * SA, SA_LAST)],
                        out.at[pl.ds(pl.multiple_of(c * N + (NS - 1) * SA, 8),
                                     SA_LAST)])


@functools.partial(
    pl.kernel,
    out_type=(
        jax.ShapeDtypeStruct((3, NC, B, H), jnp.float32),
        jax.ShapeDtypeStruct((3, B, D), jnp.float32),
    ),
    mesh=_mesh,
    scratch_types=[
        pltpu.VMEM((1, G), jnp.int32),
        pltpu.VMEM((G, H), jnp.float32),
        pltpu.VMEM((G, H), jnp.float32),
        pltpu.VMEM((G, H), jnp.float32),
        pltpu.VMEM((G, H), jnp.float32),
        pltpu.VMEM((G, H), jnp.float32),
        pltpu.VMEM((G, D), jnp.float32),
        pltpu.VMEM((16,), jnp.float32),
    ],
    compiler_params=pltpu.CompilerParams(use_tc_tiling_on_sc=False, needs_layout_passes=False),
)
def _gather_combine(e0, e1, e2, e3, ego_full, idxcat, wpad, out_f, out_e,
                    idxv, b0, b1, b2, b3, obuf, ebuf, wbuf):
    c = lax.axis_index("c")
    s = lax.axis_index("s")
    pltpu.sync_copy(wpad, wbuf)
    wraw = wbuf[...]
    m = jnp.max(wraw)
    ex = jnp.exp(wraw - lax.broadcast(m, (16,)))
    ssum = jnp.sum(ex)
    wv = ex / lax.broadcast(ssum, (16,))
    bcw = [_bcast_lane(wv, t) for t in range(4)]

    per_sub = B // NS

    def jk_body(jk, carry):
        j = jk // 2
        k = jk - 2 * j
        off = s * per_sub + k * G
        src = pl.multiple_of((c * 3 + j) * B + off, G)
        pltpu.sync_copy(idxcat.at[pl.ds(src, G)], idxv.at[0])
        pltpu.sync_copy(e0.at[idxv.at[0]], b0)
        pltpu.sync_copy(e1.at[idxv.at[0]], b1)
        pltpu.sync_copy(e2.at[idxv.at[0]], b2)
        pltpu.sync_copy(e3.at[idxv.at[0]], b3)
        for e in range(G):
            for q in range(H // 16):
                sl = pl.ds(q * 16, 16)
                obuf[e, sl] = (b0[e, sl] * bcw[0] + b1[e, sl] * bcw[1]
                               + b2[e, sl] * bcw[2] + b3[e, sl] * bcw[3])
        pltpu.sync_copy(obuf, out_f.at[j, c, pl.ds(off, G)])
        return carry

    lax.fori_loop(0, 6, jk_body, 0)

    w = s * NC + c

    def ego_body(j, carry):
        pltpu.sync_copy(idxcat.at[pl.ds(pl.multiple_of(j * B + w * G, G), G)],
                        idxv.at[0])
        pltpu.sync_copy(ego_full.at[idxv.at[0]], ebuf)
        pltpu.sync_copy(ebuf, out_e.at[j, pl.ds(w * G, G)])
        return carry

    lax.fori_loop(0, 3, ego_body, 0)


def kernel(adj_indices, adj_values, users, pos_items, neg_items,
           user_table, item_table, layer_weights):
    ego_full = jnp.concatenate([user_table, item_table], axis=0)
    ego_flat = jnp.concatenate([ego_full[:, :H], ego_full[:, H:]], axis=0)
    row = adj_indices[0]
    col = adj_indices[1]
    col2d = jnp.concatenate([col, col + N]).reshape(2 * NCHUNKS, G)
    row2d = row.reshape(NCHUNKS, G)
    val2d = adj_values.reshape(NCHUNKS, G)
    zeros = jnp.zeros((N, H), jnp.float32)
    wpad = jnp.concatenate(
        [layer_weights, jnp.full((12,), -1e30, jnp.float32)])
    iu = users
    ip = N_U + pos_items
    ineg = N_U + neg_items
    idxcat = jnp.concatenate(
        [iu, ip, ineg, iu + N, ip + N, ineg + N]).astype(jnp.int32)

    e1 = _prop(ego_flat, col2d, row2d, val2d, zeros)
    e2 = _prop(e1, col2d, row2d, val2d, zeros)
    e3 = _prop(e2, col2d, row2d, val2d, zeros)

    out_f, out_e = _gather_combine(ego_flat, e1, e2, e3, ego_full,
                                   idxcat, wpad)
    ue = jnp.concatenate([out_f[0, 0], out_f[0, 1]], axis=1)
    pe = jnp.concatenate([out_f[1, 0], out_f[1, 1]], axis=1)
    ne = jnp.concatenate([out_f[2, 0], out_f[2, 1]], axis=1)
    return ue, pe, ne, out_e[0], out_e[1], out_e[2]
